# baseline (speedup 1.0000x reference)
import functools

import jax
import jax.numpy as jnp
from jax import lax
from jax.experimental import pallas as pl
from jax.experimental.pallas import tpu as pltpu
from jax.experimental.pallas import tpu_sc as plsc

B = 16384
V2 = 500000
HIST = 50
D = 64
DP = 128
H = 64

NC = 2
NS = 16
NW = NC * NS
ROWS_W = B // NW
SPC = 8
TPC = SPC * HIST
CPW = ROWS_W // SPC
TOK_W = ROWS_W * HIST
QTOK = TOK_W // 4
NB = 2


def _pooled_sums(x1, emb2):
    mesh = plsc.VectorSubcoreMesh(
        core_axis_name="c", subcore_axis_name="s", num_cores=NC, num_subcores=NS
    )

    @functools.partial(
        pl.kernel,
        out_type=jax.ShapeDtypeStruct((B, DP), jnp.float32),
        mesh=mesh,
        compiler_params=pltpu.CompilerParams(use_tc_tiling_on_sc=True),
        scratch_types=[
            pltpu.VMEM((2 * QTOK,), jnp.int32),
            [pltpu.VMEM((TPC, DP), jnp.float32) for _ in range(NB)],
            [pltpu.VMEM((SPC, DP), jnp.float32) for _ in range(NB)],
            [pltpu.SemaphoreType.DMA for _ in range(NB)],
            pltpu.SemaphoreType.DMA,
        ],
    )
    def pool(x1_hbm, emb_hbm, out_hbm, idx_all, gbufs, outcs, sems, osem):
        wid = lax.axis_index("s") * NC + lax.axis_index("c")

        def load_quarter(q):
            pltpu.sync_copy(
                x1_hbm.at[pl.ds(wid * TOK_W + q * QTOK, QTOK)],
                idx_all.at[pl.ds(lax.rem(q, 2) * QTOK, QTOK)],
            )

        def chunk_base(j):
            return (
                lax.bitwise_and(lax.shift_right_logical(j, 4), 1) * QTOK
                + lax.bitwise_and(j, 15) * TPC
            )

        load_quarter(0)

        def start_gathers(j, gb, sem):
            base = chunk_base(j)
            for c in range(TPC // 16):
                tok = idx_all[pl.ds(base + c * 16, 16)]
                iv = jnp.where(tok >= V2, tok - V2, tok)
                pltpu.async_copy(emb_hbm.at[iv], gb.at[pl.ds(c * 16, 16)], sem)

        def wait_gathers(gb, sem):
            dummy = jnp.zeros((16,), jnp.int32)
            for c in range(TPC // 16):
                pltpu.make_async_copy(
                    emb_hbm.at[dummy], gb.at[pl.ds(c * 16, 16)], sem
                ).wait()

        for p in range(NB - 1):
            start_gathers(p, gbufs[p], sems[p])

        def body(jj, carry):
            for b in range(NB):
                j = jj * NB + b

                nxt = j + NB - 1

                @pl.when(nxt < CPW)
                def _start_next():
                    @pl.when(lax.bitwise_and(nxt, 15) == 0)
                    def _reload():
                        load_quarter(lax.shift_right_logical(nxt, 4))

                    s = (b + NB - 1) % NB
                    start_gathers(nxt, gbufs[s], sems[s])

                gb = gbufs[b]
                oc = outcs[b]
                wait_gathers(gb, sems[b])

                @pl.when(jj > 0)
                def _drain_out():
                    pltpu.make_async_copy(oc, out_hbm.at[pl.ds(0, SPC)],
                                          osem).wait()

                base = chunk_base(j)

                def sample_body(r, carry, _gb=gb, _oc=oc, _base=base):
                    sbase = _base + r * HIST
                    rbase = r * HIST
                    offs = []
                    for q, qoff in enumerate((0, 16, 32, 34)):
                        pv = jnp.where(
                            idx_all[pl.ds(sbase + qoff, 16)] >= V2, 64, 0
                        )
                        lanes = range(16) if q < 3 else range(14, 16)
                        for lane in lanes:
                            offs.append((qoff + lane, pv[lane]))
                    acc = [jnp.zeros((16,), jnp.float32) for _ in range(4)]
                    for t, off in offs:
                        for c in range(4):
                            acc[c] = acc[c] + _gb[rbase + t,
                                                  pl.ds(off + c * 16, 16)]
                    for c in range(4):
                        _oc[r, pl.ds(c * 16, 16)] = acc[c]
                    zv = jnp.zeros((16,), jnp.float32)
                    for c in range(4, 8):
                        _oc[r, pl.ds(c * 16, 16)] = zv
                    return carry

                lax.fori_loop(0, SPC, sample_body, 0)

                pltpu.async_copy(
                    oc, out_hbm.at[pl.ds(wid * ROWS_W + j * SPC, SPC)], osem
                )
            return carry

        lax.fori_loop(0, CPW // NB, body, 0)

        for b in range(NB):
            pltpu.make_async_copy(outcs[b], out_hbm.at[pl.ds(0, SPC)],
                                  osem).wait()

    return pool(x1, emb2)


BT = 2048


def _mlp_body(p_ref, w1_ref, b1_ref, w2_ref, b2_ref, o_ref):
    p = p_ref[...][:, :D] * (1.0 / HIST)
    h = lax.dot_general(
        p, w1_ref[...], (((1,), (1,)), ((), ())), preferred_element_type=jnp.float32
    )
    h = jnp.maximum(h + b1_ref[...], 0.0)
    o = jnp.sum(h * w2_ref[...], axis=1, keepdims=True)
    o_ref[...] = o + b2_ref[0, 0]


def _mlp(pooled, W1, b1, W2, b2):
    return pl.pallas_call(
        _mlp_body,
        grid=(B // BT,),
        in_specs=[
            pl.BlockSpec((BT, DP), lambda i: (i, 0)),
            pl.BlockSpec((H, D), lambda i: (0, 0)),
            pl.BlockSpec((1, H), lambda i: (0, 0)),
            pl.BlockSpec((1, H), lambda i: (0, 0)),
            pl.BlockSpec(memory_space=pltpu.SMEM),
        ],
        out_specs=pl.BlockSpec((BT, 1), lambda i: (i, 0)),
        out_shape=jax.ShapeDtypeStruct((B, 1), jnp.float32),
    )(pooled, W1, b1.reshape(1, H), W2, b2.reshape(1, 1))


def kernel(x, emb, W1, b1, W2, b2):
    x1 = x.astype(jnp.int32).reshape(B * HIST)
    emb2 = jnp.concatenate([emb[:V2], emb[V2:]], axis=1)
    pooled = _pooled_sums(x1, emb2)
    return _mlp(pooled, W1, b1, W2, b2)


# speedup vs baseline: 1.1842x; 1.1842x over previous
import functools

import jax
import jax.numpy as jnp
from jax import lax
from jax.experimental import pallas as pl
from jax.experimental.pallas import tpu as pltpu
from jax.experimental.pallas import tpu_sc as plsc

B = 16384
V2 = 500000
HIST = 50
D = 64
DP = 128
H = 64

NC = 2
NS = 16
NW = NC * NS
ROWS_W = B // NW
SPC = 8
TPC = SPC * HIST
CPW = ROWS_W // SPC
TOK_W = ROWS_W * HIST
QTOK = TOK_W // 4
NB = 2


def _pooled_sums(x1, emb2):
    mesh = plsc.VectorSubcoreMesh(
        core_axis_name="c", subcore_axis_name="s", num_cores=NC, num_subcores=NS
    )

    @functools.partial(
        pl.kernel,
        out_type=jax.ShapeDtypeStruct((B, DP), jnp.float32),
        mesh=mesh,
        compiler_params=pltpu.CompilerParams(use_tc_tiling_on_sc=True),
        scratch_types=[
            pltpu.VMEM((2 * QTOK,), jnp.int32),
            [pltpu.VMEM((TPC, DP), jnp.float32) for _ in range(NB)],
            [pltpu.VMEM((SPC, DP), jnp.float32) for _ in range(NB)],
            [pltpu.SemaphoreType.DMA for _ in range(NB)],
            pltpu.SemaphoreType.DMA,
        ],
    )
    def pool(x1_hbm, emb_hbm, out_hbm, idx_all, gbufs, outcs, sems, osem):
        wid = lax.axis_index("s") * NC + lax.axis_index("c")

        def load_quarter(q):
            pltpu.sync_copy(
                x1_hbm.at[pl.ds(wid * TOK_W + q * QTOK, QTOK)],
                idx_all.at[pl.ds(lax.rem(q, 2) * QTOK, QTOK)],
            )

        def chunk_base(j):
            return (
                lax.bitwise_and(lax.shift_right_logical(j, 4), 1) * QTOK
                + lax.bitwise_and(j, 15) * TPC
            )

        load_quarter(0)

        def start_gathers(j, gb, sem):
            base = chunk_base(j)
            for c in range(TPC // 16):
                tok = idx_all[pl.ds(base + c * 16, 16)]
                iv = lax.shift_right_logical(tok, 1)
                pltpu.async_copy(emb_hbm.at[iv], gb.at[pl.ds(c * 16, 16)], sem)

        def wait_gathers(gb, sem):
            dummy = jnp.zeros((16,), jnp.int32)
            for c in range(TPC // 16):
                pltpu.make_async_copy(
                    emb_hbm.at[dummy], gb.at[pl.ds(c * 16, 16)], sem
                ).wait()

        for p in range(NB - 1):
            start_gathers(p, gbufs[p], sems[p])

        def body(jj, carry):
            for b in range(NB):
                j = jj * NB + b

                nxt = j + NB - 1

                @pl.when(nxt < CPW)
                def _start_next():
                    @pl.when(lax.bitwise_and(nxt, 15) == 0)
                    def _reload():
                        load_quarter(lax.shift_right_logical(nxt, 4))

                    s = (b + NB - 1) % NB
                    start_gathers(nxt, gbufs[s], sems[s])

                gb = gbufs[b]
                oc = outcs[b]
                wait_gathers(gb, sems[b])

                @pl.when(jj > 0)
                def _drain_out():
                    pltpu.make_async_copy(oc, out_hbm.at[pl.ds(0, SPC)],
                                          osem).wait()

                base = chunk_base(j)

                def sample_body(r, carry, _gb=gb, _oc=oc, _base=base):
                    sbase = _base + r * HIST
                    rbase = r * HIST
                    offs = []
                    for q, qoff in enumerate((0, 16, 32, 34)):
                        pv = lax.shift_left(
                            lax.bitwise_and(
                                idx_all[pl.ds(sbase + qoff, 16)], 1
                            ),
                            6,
                        )
                        lanes = range(16) if q < 3 else range(14, 16)
                        for lane in lanes:
                            offs.append((qoff + lane, pv[lane]))
                    acc = [jnp.zeros((16,), jnp.float32) for _ in range(4)]
                    for t, off in offs:
                        for c in range(4):
                            acc[c] = acc[c] + _gb[rbase + t,
                                                  pl.ds(off + c * 16, 16)]
                    for c in range(4):
                        _oc[r, pl.ds(c * 16, 16)] = acc[c]
                    zv = jnp.zeros((16,), jnp.float32)
                    for c in range(4, 8):
                        _oc[r, pl.ds(c * 16, 16)] = zv
                    return carry

                lax.fori_loop(0, SPC, sample_body, 0)

                pltpu.async_copy(
                    oc, out_hbm.at[pl.ds(wid * ROWS_W + j * SPC, SPC)], osem
                )
            return carry

        lax.fori_loop(0, CPW // NB, body, 0)

        for b in range(NB):
            pltpu.make_async_copy(outcs[b], out_hbm.at[pl.ds(0, SPC)],
                                  osem).wait()

    return pool(x1, emb2)


BT = 2048


def _mlp_body(p_ref, w1_ref, b1_ref, w2_ref, b2_ref, o_ref):
    p = p_ref[...][:, :D] * (1.0 / HIST)
    h = lax.dot_general(
        p, w1_ref[...], (((1,), (1,)), ((), ())), preferred_element_type=jnp.float32
    )
    h = jnp.maximum(h + b1_ref[...], 0.0)
    o = jnp.sum(h * w2_ref[...], axis=1, keepdims=True)
    o_ref[...] = o + b2_ref[0, 0]


def _mlp(pooled, W1, b1, W2, b2):
    return pl.pallas_call(
        _mlp_body,
        grid=(B // BT,),
        in_specs=[
            pl.BlockSpec((BT, DP), lambda i: (i, 0)),
            pl.BlockSpec((H, D), lambda i: (0, 0)),
            pl.BlockSpec((1, H), lambda i: (0, 0)),
            pl.BlockSpec((1, H), lambda i: (0, 0)),
            pl.BlockSpec(memory_space=pltpu.SMEM),
        ],
        out_specs=pl.BlockSpec((BT, 1), lambda i: (i, 0)),
        out_shape=jax.ShapeDtypeStruct((B, 1), jnp.float32),
    )(pooled, W1, b1.reshape(1, H), W2, b2.reshape(1, 1))


def kernel(x, emb, W1, b1, W2, b2):
    x1 = x.astype(jnp.int32).reshape(B * HIST)
    emb2 = emb.reshape(2 * V2 * D).reshape(V2, DP)
    pooled = _pooled_sums(x1, emb2)
    return _mlp(pooled, W1, b1, W2, b2)
